# baseline (speedup 1.0000x reference)
import functools

import jax
import jax.numpy as jnp
from jax import lax
from jax.experimental import pallas as pl
from jax.experimental.pallas import tpu as pltpu
from jax.experimental.pallas import tpu_sc as plsc

CA, CB = 96, 104
K = 4


@functools.lru_cache(maxsize=None)
def _build(n_rows: int, n_cols: int, emb: int, nc: int, ns: int):
    nw = nc * ns
    assert n_rows % nw == 0 and n_cols == CA + CB
    rows_per_w = n_rows // nw
    nchunks = rows_per_w * 2
    assert nchunks % (2 * K) == 0
    ngroups = nchunks // K
    npairs = (ngroups - 2) // 2

    mesh = plsc.VectorSubcoreMesh(core_axis_name="c", subcore_axis_name="s")

    @functools.partial(
        pl.kernel,
        out_type=jax.ShapeDtypeStruct((n_rows, n_cols, emb), jnp.float32),
        mesh=mesh,
        scratch_types=[
            pltpu.VMEM((rows_per_w, n_cols), jnp.int32),
            pltpu.VMEM((2 * K, CB, emb), jnp.float32),
            pltpu.SemaphoreType.DMA,
            pltpu.SemaphoreType.DMA,
            pltpu.SemaphoreType.DMA,
            pltpu.SemaphoreType.DMA,
        ],
        compiler_params=pltpu.CompilerParams(use_tc_tiling_on_sc=False),
    )
    def emb_kernel(table_hbm, idx_hbm, out_hbm, idx_v, rows_v,
                   gsem0, gsem1, osem0, osem1):
        wid = lax.axis_index("s") * nc + lax.axis_index("c")
        row0 = wid * rows_per_w
        pltpu.sync_copy(idx_hbm.at[pl.ds(row0, rows_per_w)], idx_v)
        gsem = (gsem0, gsem1)
        osem = (osem0, osem1)

        def _cw(b):
            return (0, CA) if b % 2 == 0 else (CA, CB)

        def fire_gathers(g, h):
            for b in range(K):
                c0, cw = _cw(b)
                pltpu.async_copy(
                    table_hbm.at[idx_v.at[2 * g + b // 2, pl.ds(c0, cw)]],
                    rows_v.at[h * K + b, pl.ds(0, cw)], gsem[h])

        def wait_gathers(h):
            for b in range(K):
                c0, cw = _cw(b)
                pltpu.make_async_copy(
                    table_hbm.at[idx_v.at[0, pl.ds(0, cw)]],
                    rows_v.at[h * K + b, pl.ds(0, cw)], gsem[h]).wait()

        def fire_outs(g, h):
            for b in range(K):
                c0, cw = _cw(b)
                pltpu.async_copy(
                    rows_v.at[h * K + b, pl.ds(0, cw)],
                    out_hbm.at[row0 + 2 * g + b // 2, pl.ds(c0, cw)],
                    osem[h])

        def wait_outs(h):
            for b in range(K):
                c0, cw = _cw(b)
                pltpu.make_async_copy(
                    rows_v.at[h * K + b, pl.ds(0, cw)],
                    out_hbm.at[row0, pl.ds(c0, cw)], osem[h]).wait()

        fire_gathers(0, 0)
        wait_gathers(0)
        fire_outs(0, 0)
        fire_gathers(1, 1)

        def pair_body(t, carry):
            g1 = 2 * t + 1
            wait_gathers(1)
            fire_outs(g1, 1)
            wait_outs(0)
            fire_gathers(g1 + 1, 0)
            wait_gathers(0)
            fire_outs(g1 + 1, 0)
            wait_outs(1)
            fire_gathers(g1 + 2, 1)
            return carry

        lax.fori_loop(0, npairs, pair_body, 0)

        wait_gathers(1)
        fire_outs(ngroups - 1, 1)
        wait_outs(0)
        wait_outs(1)

    return emb_kernel


def kernel(indices, table):
    n_rows, n_cols = indices.shape
    emb = table.shape[1]
    info = plsc.get_sparse_core_info()
    emb_kernel = _build(n_rows, n_cols, emb, info.num_cores, info.num_subcores)
    return emb_kernel(table, indices)
